# speedup vs baseline: 2.7044x; 2.7044x over previous
import jax
import jax.numpy as jnp
from jax.experimental import pallas as pl

N_NODES = 100000
N_GRAPHS = 512
BASE_DIM = 128
CHARGE_PAD = 128
NB = 2000


def _fused_kernel(sp_ref, ch_ref, bt_ref, ga_ref, ct_ref,
                  cW1_ref, cb1_ref, cW2_ref, cb2_ref,
                  pW1_ref, pb1_ref, pW2_ref, pb2_ref, out_ref):
    f32 = jnp.float32
    pW1_s = pW1_ref[:BASE_DIM]
    pW1_c = pW1_ref[BASE_DIM:BASE_DIM + 16]
    pW1_h = pW1_ref[BASE_DIM + 16:]

    C = jnp.dot(ct_ref[...], pW1_c, preferred_element_type=f32)
    h = jnp.dot(ga_ref[...], cW1_ref[...], preferred_element_type=f32) + cb1_ref[...]
    h = h * jax.nn.sigmoid(h)
    h = jnp.dot(h, cW2_ref[...], preferred_element_type=f32) + cb2_ref[...]
    G = jnp.dot(h, pW1_h, preferred_element_type=f32)

    z = jnp.dot(sp_ref[...], pW1_s, preferred_element_type=f32)
    oh_c = (ch_ref[...] == jax.lax.broadcasted_iota(jnp.int32, (NB, CHARGE_PAD), 1)).astype(f32)
    z = z + jnp.dot(oh_c, C, preferred_element_type=f32)
    oh_b = (bt_ref[...] == jax.lax.broadcasted_iota(jnp.int32, (NB, N_GRAPHS), 1)).astype(f32)
    z = z + jnp.dot(oh_b, G, preferred_element_type=f32)
    z = z + pb1_ref[...]
    a = z * jax.nn.sigmoid(z)
    out_ref[...] = jnp.dot(a, pW2_ref[...], preferred_element_type=f32) + pb2_ref[...]


def kernel(species_emb, batch, charge, graph_attr, charge_table,
           cW1, cb1, cW2, cb2, pW1, pb1, pW2, pb2):
    n = species_emb.shape[0]
    grid = n // NB
    ch2d = charge.astype(jnp.int32).reshape(n, 1)
    bt2d = batch.astype(jnp.int32).reshape(n, 1)
    ct_pad = jnp.zeros((CHARGE_PAD, charge_table.shape[1]), jnp.float32).at[:charge_table.shape[0]].set(charge_table)

    full = lambda s: pl.BlockSpec(s, lambda i: (0, 0))
    out = pl.pallas_call(
        _fused_kernel,
        grid=(grid,),
        in_specs=[
            pl.BlockSpec((NB, BASE_DIM), lambda i: (i, 0)),
            pl.BlockSpec((NB, 1), lambda i: (i, 0)),
            pl.BlockSpec((NB, 1), lambda i: (i, 0)),
            full(graph_attr.shape),
            full(ct_pad.shape),
            full(cW1.shape),
            full((1, cb1.shape[0])),
            full(cW2.shape),
            full((1, cb2.shape[0])),
            full(pW1.shape),
            full((1, pb1.shape[0])),
            full(pW2.shape),
            full((1, pb2.shape[0])),
        ],
        out_specs=pl.BlockSpec((NB, pW2.shape[1]), lambda i: (i, 0)),
        out_shape=jax.ShapeDtypeStruct((n, pW2.shape[1]), jnp.float32),
    )(species_emb, ch2d, bt2d, graph_attr, ct_pad,
      cW1, cb1.reshape(1, -1), cW2, cb2.reshape(1, -1),
      pW1, pb1.reshape(1, -1), pW2, pb2.reshape(1, -1))
    return out
